# baseline (speedup 1.0000x reference)
import jax
import jax.numpy as jnp
from jax.experimental import pallas as pl

_ROWS = 100000
_BLOCK = 11112


def _copy2_body(sr_ref, tg_ref, sr_out, tg_out):
    sr_out[...] = sr_ref[...]
    tg_out[...] = tg_ref[...]


def kernel(sr_table, tg_table, rel_table, W, b):
    grid = (pl.cdiv(_ROWS, _BLOCK),)
    spec = pl.BlockSpec((_BLOCK, 128), lambda i: (i, 0))
    out = pl.pallas_call(
        _copy2_body,
        grid=grid,
        in_specs=[spec, spec],
        out_specs=[spec, spec],
        out_shape=[
            jax.ShapeDtypeStruct(sr_table.shape, sr_table.dtype),
            jax.ShapeDtypeStruct(tg_table.shape, tg_table.dtype),
        ],
    )(sr_table, tg_table)
    return (out[0], out[1])


# speedup vs baseline: 1.0173x; 1.0173x over previous
import jax
import jax.numpy as jnp
from jax.experimental import pallas as pl

_ROWS = 100000
_BLOCK = 13336


def _copy2_body(sr_ref, tg_ref, sr_out, tg_out):
    sr_out[...] = sr_ref[...]
    tg_out[...] = tg_ref[...]


def kernel(sr_table, tg_table, rel_table, W, b):
    grid = (pl.cdiv(_ROWS, _BLOCK),)
    spec = pl.BlockSpec((_BLOCK, 128), lambda i: (i, 0))
    out = pl.pallas_call(
        _copy2_body,
        grid=grid,
        in_specs=[spec, spec],
        out_specs=[spec, spec],
        out_shape=[
            jax.ShapeDtypeStruct(sr_table.shape, sr_table.dtype),
            jax.ShapeDtypeStruct(tg_table.shape, tg_table.dtype),
        ],
    )(sr_table, tg_table)
    return (out[0], out[1])
